# baseline (speedup 1.0000x reference)
import functools

import jax
import jax.numpy as jnp
from jax import lax
from jax.experimental import pallas as pl
from jax.experimental.pallas import tpu as pltpu
from jax.experimental.pallas import tpu_sc as plsc

N = 10000
E = 320000
D = 128
H = 8
HD = 16

NC = 2
NS = 16
NW = NC * NS
C = 80
K = E // (NW * C)
NP = 10112
RPS = NP // NS


def _qkv_body(x_ref, w_ref, b_ref, q_ref, kv_ref):
    acc = jnp.dot(x_ref[...], w_ref[...],
                  preferred_element_type=jnp.float32) + b_ref[...]

    def pack(hi, lo):
        h = lax.bitcast_convert_type(
            hi.astype(jnp.bfloat16), jnp.uint16).astype(jnp.uint32) << 16
        l = lax.bitcast_convert_type(
            lo.astype(jnp.bfloat16), jnp.uint16).astype(jnp.uint32)
        return lax.bitcast_convert_type(h | l, jnp.float32)

    q_ref[...] = acc[:, :D]
    kv_ref[...] = pack(acc[:, D:2 * D], acc[:, 2 * D:])


def _qkv_call(x, Wc, bc):
    BN = 2000
    g = N // BN
    return pl.pallas_call(
        _qkv_body,
        grid=(g,),
        in_specs=[
            pl.BlockSpec((BN, D), lambda i: (i, 0)),
            pl.BlockSpec((D, 3 * D), lambda i: (0, 0)),
            pl.BlockSpec((1, 3 * D), lambda i: (0, 0)),
        ],
        out_specs=[
            pl.BlockSpec((BN, D), lambda i: (i, 0)),
            pl.BlockSpec((BN, D), lambda i: (i, 0)),
        ],
        out_shape=[jax.ShapeDtypeStruct((N, D), jnp.float32)] * 2,
    )(x, Wc, bc)


def _gather2_body(kvh, qh, src2, dst2, kvg, qdg,
                  si0, si1, di0, di1, kvb0, kvb1, qb0, qb1,
                  sx0, sx1, sg0, sg1, sw0, sw1):
    c = lax.axis_index("c")
    s = lax.axis_index("s")
    w = c * NS + s
    sis = (si0, si1)
    dis = (di0, di1)
    kvbs = (kvb0, kvb1)
    qbs = (qb0, qb1)
    sxs = (sx0, sx1)
    sgs = (sg0, sg1)
    sws = (sw0, sw1)

    def start_idx(j, b):
        r = w * K + j
        pltpu.async_copy(src2.at[r, 0], sis[b], sxs[b])
        pltpu.async_copy(dst2.at[r, 0], dis[b], sxs[b])

    def drain_idx(b):
        pltpu.make_async_copy(src2.at[w * K, 0], sis[b], sxs[b]).wait()
        pltpu.make_async_copy(src2.at[w * K, 0], dis[b], sxs[b]).wait()

    def drain_wb(b):
        pltpu.make_async_copy(kvg.at[pl.ds(0, C)], kvbs[b], sws[b]).wait()
        pltpu.make_async_copy(qdg.at[pl.ds(0, C)], qbs[b], sws[b]).wait()

    def fire(j, b):
        r = w * K + j
        drain_idx(b)

        @pl.when(j >= 2)
        def _():
            drain_wb(b)

        ck = pltpu.async_copy(kvh.at[sis[b]], kvbs[b], sgs[b])
        cq = pltpu.async_copy(qh.at[dis[b]], qbs[b], sgs[b])
        ck.wait()
        cq.wait()

        @pl.when(j + 2 < K)
        def _():
            start_idx(j + 2, b)

        base = r * C
        pltpu.async_copy(kvbs[b], kvg.at[pl.ds(base, C)], sws[b])
        pltpu.async_copy(qbs[b], qdg.at[pl.ds(base, C)], sws[b])

    start_idx(0, 0)
    start_idx(1, 1)

    def body(g, carry):
        fire(2 * g, 0)
        fire(2 * g + 1, 1)
        return carry

    lax.fori_loop(0, (K - 1) // 2, body, 0)
    fire(K - 1, (K - 1) % 2)
    drain_wb(0)
    drain_wb(1)


def _gather2_call(kvh, qh, src2, dst2):
    mesh = plsc.VectorSubcoreMesh(core_axis_name="c", subcore_axis_name="s")
    fn = functools.partial(
        pl.kernel,
        out_type=[jax.ShapeDtypeStruct((E, D), jnp.float32)] * 2,
        mesh=mesh,
        scratch_types=[
            pltpu.VMEM((C,), jnp.int32),
            pltpu.VMEM((C,), jnp.int32),
            pltpu.VMEM((C,), jnp.int32),
            pltpu.VMEM((C,), jnp.int32),
            pltpu.VMEM((C, D), jnp.float32),
            pltpu.VMEM((C, D), jnp.float32),
            pltpu.VMEM((C, D), jnp.float32),
            pltpu.VMEM((C, D), jnp.float32),
            pltpu.SemaphoreType.DMA,
            pltpu.SemaphoreType.DMA,
            pltpu.SemaphoreType.DMA,
            pltpu.SemaphoreType.DMA,
            pltpu.SemaphoreType.DMA,
            pltpu.SemaphoreType.DMA,
        ],
    )(_gather2_body)
    return fn(kvh, qh, src2, dst2)


def _edge_body(ea_ref, kv_ref, qd_ref, Ew_ref, Eb_ref, Awm_ref,
               Bm_ref, Eow_ref, Eob_ref, R_ref, w_ref, S_ref, ehp_ref,
               st_ref):
    i = pl.program_id(0)
    eav = ea_ref[...]
    f = jnp.dot(eav.astype(jnp.bfloat16), Ew_ref[...],
                preferred_element_type=jnp.float32) + Eb_ref[...]
    a = f[:, :D]
    b = f[:, D:]
    s2 = a * b
    p = jnp.sign(s2) * jnp.sqrt(jnp.abs(s2))
    def unpack(pk):
        u = lax.bitcast_convert_type(pk, jnp.uint32)
        hi = lax.bitcast_convert_type(
            (u >> 16).astype(jnp.uint16), jnp.bfloat16).astype(jnp.float32)
        lo = lax.bitcast_convert_type(
            u.astype(jnp.uint16), jnp.bfloat16).astype(jnp.float32)
        return hi, lo

    ks, vs = unpack(kv_ref[...])
    score = jax.nn.relu(ks + qd_ref[...] + p)
    score_b = score.astype(jnp.bfloat16)
    logits = jnp.dot(score, Awm_ref[...], preferred_element_type=jnp.float32)
    wv = jnp.exp(jnp.clip(logits, -5.0, 5.0))
    wex = jnp.dot(wv, R_ref[...], preferred_element_type=jnp.float32)
    w_ref[...] = wex
    G = jnp.dot(score_b, Bm_ref[...], preferred_element_type=jnp.float32)
    S_ref[...] = (vs + G) * wex
    ehp = eav + jnp.dot(score_b, Eow_ref[...],
                        preferred_element_type=jnp.float32) + Eob_ref[...]
    ehp_ref[...] = ehp.astype(jnp.bfloat16)

    @pl.when(i == 0)
    def _():
        st_ref[...] = jnp.zeros_like(st_ref)

    s1 = jnp.sum(ehp, axis=0, keepdims=True)
    sq = jnp.sum(ehp * ehp, axis=0, keepdims=True)
    st_ref[...] += jnp.concatenate(
        [s1, sq, jnp.zeros((6, D), jnp.float32)], axis=0)


def _edge_call(ea, kvg, qdg, Ew2, Eb2, Awm, Bm, Eow, Eob2, R):
    BE = 2000
    g = E // BE
    full = lambda shape: pl.BlockSpec(shape, lambda i: (0, 0))
    return pl.pallas_call(
        _edge_body,
        grid=(g,),
        in_specs=[
            pl.BlockSpec((BE, D), lambda i: (i, 0)),
            pl.BlockSpec((BE, D), lambda i: (i, 0)),
            pl.BlockSpec((BE, D), lambda i: (i, 0)),
            full((D, 2 * D)),
            full((1, 2 * D)),
            full((D, H)),
            full((D, D)),
            full((D, D)),
            full((1, D)),
            full((H, D)),
        ],
        out_specs=[
            pl.BlockSpec((BE, D), lambda i: (i, 0)),
            pl.BlockSpec((BE, D), lambda i: (i, 0)),
            pl.BlockSpec((BE, D), lambda i: (i, 0)),
            pl.BlockSpec((8, D), lambda i: (0, 0)),
        ],
        out_shape=[
            jax.ShapeDtypeStruct((E, D), jnp.float32),
            jax.ShapeDtypeStruct((E, D), jnp.float32),
            jax.ShapeDtypeStruct((E, D), jnp.bfloat16),
            jax.ShapeDtypeStruct((8, D), jnp.float32),
        ],
    )(ea, kvg, qdg, Ew2, Eb2, Awm, Bm, Eow, Eob2, R)


def _scatter_body(w_hbm, S_hbm, dst2, z128, sout, vout,
                  di0, di1, sv0, sv1, acc, si0, si1, sd0, sd1):
    c = lax.axis_index("c")
    s = lax.axis_index("s")
    w = c * NS + s
    dis = (di0, di1)
    svs = (sv0, sv1)
    sis = (si0, si1)
    sds = (sd0, sd1)

    def one_pass(src_hbm, out_hbm):
        pltpu.sync_copy(z128, acc.at[pl.ds(s * RPS, RPS)])
        plsc.subcore_barrier()

        def start(j, b):
            r = w * K + j
            pltpu.async_copy(dst2.at[r, 0], dis[b], sis[b])
            pltpu.async_copy(src_hbm.at[pl.ds(r * C, C)], svs[b], sds[b])

        def fire(j, b):
            pltpu.make_async_copy(dst2.at[w * K, 0], dis[b], sis[b]).wait()
            pltpu.make_async_copy(
                src_hbm.at[pl.ds(0, C)], svs[b], sds[b]).wait()
            pltpu.sync_copy(svs[b], acc.at[dis[b]], add=True)

        start(0, 0)
        start(1, 1)

        def body(g, carry):
            for b in range(2):
                j = g + b
                fire(j, b)

                @pl.when(j + 2 < K)
                def _():
                    start(j + 2, b)
            return carry

        lax.fori_loop(0, (K - 1) // 2, lambda g, cy: body(2 * g, cy), 0)
        fire(K - 1, (K - 1) % 2)
        plsc.subcore_barrier()
        pltpu.sync_copy(acc.at[pl.ds(s * RPS, RPS)],
                        out_hbm.at[c, pl.ds(s * RPS, RPS)])
        plsc.subcore_barrier()

    one_pass(S_hbm, vout)
    one_pass(w_hbm, sout)


def _scatter_call(w_e, S_e, dst2, z128):
    mesh = plsc.VectorSubcoreMesh(core_axis_name="c", subcore_axis_name="s")
    fn = functools.partial(
        pl.kernel,
        out_type=[
            jax.ShapeDtypeStruct((NC, NP, D), jnp.float32),
            jax.ShapeDtypeStruct((NC, NP, D), jnp.float32),
        ],
        mesh=mesh,
        scratch_types=[
            pltpu.VMEM((C,), jnp.int32),
            pltpu.VMEM((C,), jnp.int32),
            pltpu.VMEM((C, D), jnp.float32),
            pltpu.VMEM((C, D), jnp.float32),
            pltpu.VMEM_SHARED((NP, D), jnp.float32),
            pltpu.SemaphoreType.DMA,
            pltpu.SemaphoreType.DMA,
            pltpu.SemaphoreType.DMA,
            pltpu.SemaphoreType.DMA,
        ],
    )(_scatter_body)
    return fn(w_e, S_e, dst2, z128)


def _node_body(ov0_ref, ov1_ref, sp0_ref, sp1_ref, x_ref, ld_ref,
               dc0_ref, dc1_ref, Now_ref, Nob_ref, g1_ref, b1_ref,
               F1w_ref, F1b_ref, F2w_ref, F2b_ref, g2_ref, b2_ref, out_ref):
    sex = 1.0 / (sp0_ref[...] + sp1_ref[...] + 1e-16)
    oV = (ov0_ref[...] + ov1_ref[...]) * sex
    nh = oV * dc0_ref[...] + (oV * ld_ref[...]) * dc1_ref[...]
    nh = jnp.dot(nh, Now_ref[...],
                 preferred_element_type=jnp.float32) + Nob_ref[...]
    nh = x_ref[...] + nh
    mu = jnp.mean(nh, axis=0, keepdims=True)
    var = jnp.mean((nh - mu) * (nh - mu), axis=0, keepdims=True)
    nh = g1_ref[...] * (nh - mu) / jnp.sqrt(var + 1e-5) + b1_ref[...]
    nr2 = nh
    hid = jax.nn.relu(
        jnp.dot(nh, F1w_ref[...], preferred_element_type=jnp.float32)
        + F1b_ref[...])
    nh = jnp.dot(hid, F2w_ref[...],
                 preferred_element_type=jnp.float32) + F2b_ref[...]
    nh = nr2 + nh
    mu2 = jnp.mean(nh, axis=0, keepdims=True)
    var2 = jnp.mean((nh - mu2) * (nh - mu2), axis=0, keepdims=True)
    out_ref[...] = g2_ref[...] * (nh - mu2) / jnp.sqrt(var2 + 1e-5) \
        + b2_ref[...]


def _node_call(ov0, ov1, sp0, sp1, x, ld, dc0, dc1, Now, Nob, g1, b1,
               F1w, F1b, F2w, F2b, g2, b2):
    full = lambda a: pl.BlockSpec(a.shape, lambda: tuple(0 for _ in a.shape))
    args = (ov0, ov1, sp0, sp1, x, ld, dc0, dc1, Now, Nob, g1, b1,
            F1w, F1b, F2w, F2b, g2, b2)
    return pl.pallas_call(
        _node_body,
        in_specs=[full(a) for a in args],
        out_specs=pl.BlockSpec((N, D), lambda: (0, 0)),
        out_shape=jax.ShapeDtypeStruct((N, D), jnp.float32),
    )(*args)


def _ehf_body(ehp_ref, st_ref, g_ref, b_ref, out_ref):
    mu = st_ref[0:1, :] / E
    ex2 = st_ref[1:2, :] / E
    var = ex2 - mu * mu
    out_ref[...] = g_ref[...] * (ehp_ref[...].astype(jnp.float32) - mu) \
        * lax.rsqrt(var + 1e-5) + b_ref[...]


def _ehf_call(ehp, st, g, b):
    BE = 4000
    gr = E // BE
    return pl.pallas_call(
        _ehf_body,
        grid=(gr,),
        in_specs=[
            pl.BlockSpec((BE, D), lambda i: (i, 0)),
            pl.BlockSpec((8, D), lambda i: (0, 0)),
            pl.BlockSpec((1, D), lambda i: (0, 0)),
            pl.BlockSpec((1, D), lambda i: (0, 0)),
        ],
        out_specs=pl.BlockSpec((BE, D), lambda i: (i, 0)),
        out_shape=jax.ShapeDtypeStruct((E, D), jnp.float32),
    )(ehp, st, g, b)


def kernel(x, edge_attr, edge_index, log_deg, Qw, Qb, Kw, Kb, Ew, Eb, Vw,
           Vb, Aw, VeRow, deg_coef, Now, Nob, Eow, Eob, bn1n_g, bn1n_b,
           bn1e_g, bn1e_b, F1w, F1b, F2w, F2b, bn2_g, bn2_b):
    src2 = edge_index[0].reshape(NW * K, 1, C)
    dst2 = edge_index[1].reshape(NW * K, 1, C)

    Wc = jnp.concatenate([Qw, Kw, Vw], axis=1)
    bc = jnp.concatenate([Qb, Kb, Vb]).reshape(1, 3 * D)
    q, kv = _qkv_call(x, Wc, bc)

    kvg, qdg = _gather2_call(kv, q, src2, dst2)

    Ew4 = Ew.reshape(D, H, 2, HD)
    Ew2 = jnp.concatenate(
        [Ew4[:, :, 0, :].reshape(D, D), Ew4[:, :, 1, :].reshape(D, D)],
        axis=1)
    Eb4 = Eb.reshape(H, 2, HD)
    Eb2 = jnp.concatenate(
        [Eb4[:, 0, :].reshape(1, D), Eb4[:, 1, :].reshape(1, D)], axis=1)

    eye8 = jnp.eye(H, dtype=jnp.float32)
    Awm = (Aw[:, :, 0].T[:, :, None] * eye8[:, None, :]).reshape(D, H)
    Bm = (VeRow.transpose(1, 0, 2)[:, :, None, :]
          * eye8[:, None, :, None]).reshape(D, D)
    R = jnp.kron(eye8, jnp.ones((1, HD), jnp.float32))

    w_e, S_e, ehp, st = _edge_call(
        edge_attr, kvg, qdg, Ew2.astype(jnp.bfloat16), Eb2, Awm,
        Bm.astype(jnp.bfloat16), Eow.astype(jnp.bfloat16),
        Eob.reshape(1, D), R)

    z128 = jnp.zeros((RPS, D), jnp.float32)
    sout, vout = _scatter_call(w_e, S_e, dst2, z128)

    eh = _ehf_call(ehp, st, bn1e_g.reshape(1, D), bn1e_b.reshape(1, D))

    ld128 = jnp.broadcast_to(log_deg, (N, D))
    nh = _node_call(
        vout[0, :N], vout[1, :N], sout[0, :N], sout[1, :N], x, ld128,
        deg_coef[0, :, 0].reshape(1, D), deg_coef[0, :, 1].reshape(1, D),
        Now, Nob.reshape(1, D), bn1n_g.reshape(1, D), bn1n_b.reshape(1, D),
        F1w, F1b.reshape(1, 2 * D), F2w, F2b.reshape(1, D),
        bn2_g.reshape(1, D), bn2_b.reshape(1, D))

    return nh, eh


# speedup vs baseline: 1.0769x; 1.0769x over previous
import functools

import jax
import jax.numpy as jnp
from jax import lax
from jax.experimental import pallas as pl
from jax.experimental.pallas import tpu as pltpu
from jax.experimental.pallas import tpu_sc as plsc

N = 10000
E = 320000
D = 128
H = 8
HD = 16

NC = 2
NS = 16
NW = NC * NS
C = 80
K = E // (NW * C)
NP = 10112
RPS = NP // NS


def _qkv_body(x_ref, w_ref, b_ref, q_ref, kv_ref):
    acc = jnp.dot(x_ref[...], w_ref[...],
                  preferred_element_type=jnp.float32) + b_ref[...]

    def pack(hi, lo):
        h = lax.bitcast_convert_type(
            hi.astype(jnp.bfloat16), jnp.uint16).astype(jnp.uint32) << 16
        l = lax.bitcast_convert_type(
            lo.astype(jnp.bfloat16), jnp.uint16).astype(jnp.uint32)
        return lax.bitcast_convert_type(h | l, jnp.float32)

    q_ref[...] = acc[:, :D]
    kv_ref[...] = pack(acc[:, D:2 * D], acc[:, 2 * D:])


def _qkv_call(x, Wc, bc):
    BN = 2000
    g = N // BN
    return pl.pallas_call(
        _qkv_body,
        grid=(g,),
        in_specs=[
            pl.BlockSpec((BN, D), lambda i: (i, 0)),
            pl.BlockSpec((D, 3 * D), lambda i: (0, 0)),
            pl.BlockSpec((1, 3 * D), lambda i: (0, 0)),
        ],
        out_specs=[
            pl.BlockSpec((BN, D), lambda i: (i, 0)),
            pl.BlockSpec((BN, D), lambda i: (i, 0)),
        ],
        out_shape=[jax.ShapeDtypeStruct((N, D), jnp.float32)] * 2,
    )(x, Wc, bc)


def _gather2_body(kvh, qh, src2, dst2, kvg, qdg,
                  si0, si1, di0, di1, kvb0, kvb1, qb0, qb1,
                  sx0, sx1, sg0, sg1, sw0, sw1):
    c = lax.axis_index("c")
    s = lax.axis_index("s")
    w = c * NS + s
    sis = (si0, si1)
    dis = (di0, di1)
    kvbs = (kvb0, kvb1)
    qbs = (qb0, qb1)
    sxs = (sx0, sx1)
    sgs = (sg0, sg1)
    sws = (sw0, sw1)

    def start_idx(j, b):
        r = w * K + j
        pltpu.async_copy(src2.at[r, 0], sis[b], sxs[b])
        pltpu.async_copy(dst2.at[r, 0], dis[b], sxs[b])

    def drain_idx(b):
        pltpu.make_async_copy(src2.at[w * K, 0], sis[b], sxs[b]).wait()
        pltpu.make_async_copy(src2.at[w * K, 0], dis[b], sxs[b]).wait()

    def drain_wb(b):
        pltpu.make_async_copy(kvg.at[pl.ds(0, C)], kvbs[b], sws[b]).wait()
        pltpu.make_async_copy(qdg.at[pl.ds(0, C)], qbs[b], sws[b]).wait()

    def fire(j, b):
        r = w * K + j
        drain_idx(b)

        @pl.when(j >= 2)
        def _():
            drain_wb(b)

        ck = pltpu.async_copy(kvh.at[sis[b]], kvbs[b], sgs[b])
        cq = pltpu.async_copy(qh.at[dis[b]], qbs[b], sgs[b])
        ck.wait()
        cq.wait()

        @pl.when(j + 2 < K)
        def _():
            start_idx(j + 2, b)

        base = r * C
        pltpu.async_copy(kvbs[b], kvg.at[pl.ds(base, C)], sws[b])
        pltpu.async_copy(qbs[b], qdg.at[pl.ds(base, C)], sws[b])

    start_idx(0, 0)
    start_idx(1, 1)

    def body(g, carry):
        fire(2 * g, 0)
        fire(2 * g + 1, 1)
        return carry

    lax.fori_loop(0, (K - 1) // 2, body, 0)
    fire(K - 1, (K - 1) % 2)
    drain_wb(0)
    drain_wb(1)


def _gather2_call(kvh, qh, src2, dst2):
    mesh = plsc.VectorSubcoreMesh(core_axis_name="c", subcore_axis_name="s")
    fn = functools.partial(
        pl.kernel,
        out_type=[jax.ShapeDtypeStruct((E, D), jnp.float32)] * 2,
        mesh=mesh,
        scratch_types=[
            pltpu.VMEM((C,), jnp.int32),
            pltpu.VMEM((C,), jnp.int32),
            pltpu.VMEM((C,), jnp.int32),
            pltpu.VMEM((C,), jnp.int32),
            pltpu.VMEM((C, D), jnp.float32),
            pltpu.VMEM((C, D), jnp.float32),
            pltpu.VMEM((C, D), jnp.float32),
            pltpu.VMEM((C, D), jnp.float32),
            pltpu.SemaphoreType.DMA,
            pltpu.SemaphoreType.DMA,
            pltpu.SemaphoreType.DMA,
            pltpu.SemaphoreType.DMA,
            pltpu.SemaphoreType.DMA,
            pltpu.SemaphoreType.DMA,
        ],
    )(_gather2_body)
    return fn(kvh, qh, src2, dst2)


def _edge_body(ea_ref, kv_ref, qd_ref, Ew_ref, Eb_ref, Awm_ref,
               Bm_ref, Eow_ref, Eob_ref, R_ref, w_ref, S_ref, ehp_ref,
               st_ref):
    i = pl.program_id(0)
    eav = ea_ref[...]
    f = jnp.dot(eav, Ew_ref[...], preferred_element_type=jnp.float32) \
        + Eb_ref[...]
    a = f[:, :D]
    b = f[:, D:]
    s2 = a * b
    p = jnp.sign(s2) * jnp.sqrt(jnp.abs(s2))
    def unpack(pk):
        u = lax.bitcast_convert_type(pk, jnp.uint32)
        hi = lax.bitcast_convert_type(
            (u >> 16).astype(jnp.uint16), jnp.bfloat16).astype(jnp.float32)
        lo = lax.bitcast_convert_type(
            u.astype(jnp.uint16), jnp.bfloat16).astype(jnp.float32)
        return hi, lo

    ks, vs = unpack(kv_ref[...])
    score = jax.nn.relu(ks + qd_ref[...] + p)
    logits = jnp.dot(score, Awm_ref[...], preferred_element_type=jnp.float32)
    wv = jnp.exp(jnp.clip(logits, -5.0, 5.0))
    wex = jnp.dot(wv, R_ref[...], preferred_element_type=jnp.float32)
    w_ref[...] = wex
    G = jnp.dot(score, Bm_ref[...], preferred_element_type=jnp.float32)
    S_ref[...] = (vs + G) * wex
    ehp = eav + jnp.dot(score, Eow_ref[...],
                        preferred_element_type=jnp.float32) + Eob_ref[...]
    ehp_ref[...] = ehp.astype(jnp.bfloat16)

    @pl.when(i == 0)
    def _():
        st_ref[...] = jnp.zeros_like(st_ref)

    s1 = jnp.sum(ehp, axis=0, keepdims=True)
    sq = jnp.sum(ehp * ehp, axis=0, keepdims=True)
    st_ref[...] += jnp.concatenate(
        [s1, sq, jnp.zeros((6, D), jnp.float32)], axis=0)


def _edge_call(ea, kvg, qdg, Ew2, Eb2, Awm, Bm, Eow, Eob2, R):
    BE = 4000
    g = E // BE
    full = lambda shape: pl.BlockSpec(shape, lambda i: (0, 0))
    return pl.pallas_call(
        _edge_body,
        grid=(g,),
        in_specs=[
            pl.BlockSpec((BE, D), lambda i: (i, 0)),
            pl.BlockSpec((BE, D), lambda i: (i, 0)),
            pl.BlockSpec((BE, D), lambda i: (i, 0)),
            full((D, 2 * D)),
            full((1, 2 * D)),
            full((D, H)),
            full((D, D)),
            full((D, D)),
            full((1, D)),
            full((H, D)),
        ],
        out_specs=[
            pl.BlockSpec((BE, D), lambda i: (i, 0)),
            pl.BlockSpec((BE, D), lambda i: (i, 0)),
            pl.BlockSpec((BE, D), lambda i: (i, 0)),
            pl.BlockSpec((8, D), lambda i: (0, 0)),
        ],
        out_shape=[
            jax.ShapeDtypeStruct((E, D), jnp.float32),
            jax.ShapeDtypeStruct((E, D), jnp.float32),
            jax.ShapeDtypeStruct((E, D), jnp.bfloat16),
            jax.ShapeDtypeStruct((8, D), jnp.float32),
        ],
    )(ea, kvg, qdg, Ew2, Eb2, Awm, Bm, Eow, Eob2, R)


def _scatter_body(w_hbm, S_hbm, dst2, z128, sout, vout,
                  di0, di1, sv0, sv1, acc, si0, si1, sd0, sd1):
    c = lax.axis_index("c")
    s = lax.axis_index("s")
    w = c * NS + s
    dis = (di0, di1)
    svs = (sv0, sv1)
    sis = (si0, si1)
    sds = (sd0, sd1)

    def one_pass(src_hbm, out_hbm):
        pltpu.sync_copy(z128, acc.at[pl.ds(s * RPS, RPS)])
        plsc.subcore_barrier()

        def start(j, b):
            r = w * K + j
            pltpu.async_copy(dst2.at[r, 0], dis[b], sis[b])
            pltpu.async_copy(src_hbm.at[pl.ds(r * C, C)], svs[b], sds[b])

        def fire(j, b):
            pltpu.make_async_copy(dst2.at[w * K, 0], dis[b], sis[b]).wait()
            pltpu.make_async_copy(
                src_hbm.at[pl.ds(0, C)], svs[b], sds[b]).wait()
            pltpu.sync_copy(svs[b], acc.at[dis[b]], add=True)

        start(0, 0)
        start(1, 1)

        def body(g, carry):
            for b in range(2):
                j = g + b
                fire(j, b)

                @pl.when(j + 2 < K)
                def _():
                    start(j + 2, b)
            return carry

        lax.fori_loop(0, (K - 1) // 2, lambda g, cy: body(2 * g, cy), 0)
        fire(K - 1, (K - 1) % 2)
        plsc.subcore_barrier()
        pltpu.sync_copy(acc.at[pl.ds(s * RPS, RPS)],
                        out_hbm.at[c, pl.ds(s * RPS, RPS)])
        plsc.subcore_barrier()

    one_pass(S_hbm, vout)
    one_pass(w_hbm, sout)


def _scatter_call(w_e, S_e, dst2, z128):
    mesh = plsc.VectorSubcoreMesh(core_axis_name="c", subcore_axis_name="s")
    fn = functools.partial(
        pl.kernel,
        out_type=[
            jax.ShapeDtypeStruct((NC, NP, D), jnp.float32),
            jax.ShapeDtypeStruct((NC, NP, D), jnp.float32),
        ],
        mesh=mesh,
        scratch_types=[
            pltpu.VMEM((C,), jnp.int32),
            pltpu.VMEM((C,), jnp.int32),
            pltpu.VMEM((C, D), jnp.float32),
            pltpu.VMEM((C, D), jnp.float32),
            pltpu.VMEM_SHARED((NP, D), jnp.float32),
            pltpu.SemaphoreType.DMA,
            pltpu.SemaphoreType.DMA,
            pltpu.SemaphoreType.DMA,
            pltpu.SemaphoreType.DMA,
        ],
    )(_scatter_body)
    return fn(w_e, S_e, dst2, z128)


def _node_body(ov0_ref, ov1_ref, sp0_ref, sp1_ref, x_ref, ld_ref,
               dc0_ref, dc1_ref, Now_ref, Nob_ref, g1_ref, b1_ref,
               F1w_ref, F1b_ref, F2w_ref, F2b_ref, g2_ref, b2_ref, out_ref):
    sex = 1.0 / (sp0_ref[...] + sp1_ref[...] + 1e-16)
    oV = (ov0_ref[...] + ov1_ref[...]) * sex
    nh = oV * dc0_ref[...] + (oV * ld_ref[...]) * dc1_ref[...]
    nh = jnp.dot(nh, Now_ref[...],
                 preferred_element_type=jnp.float32) + Nob_ref[...]
    nh = x_ref[...] + nh
    mu = jnp.mean(nh, axis=0, keepdims=True)
    var = jnp.mean((nh - mu) * (nh - mu), axis=0, keepdims=True)
    nh = g1_ref[...] * (nh - mu) / jnp.sqrt(var + 1e-5) + b1_ref[...]
    nr2 = nh
    hid = jax.nn.relu(
        jnp.dot(nh, F1w_ref[...], preferred_element_type=jnp.float32)
        + F1b_ref[...])
    nh = jnp.dot(hid, F2w_ref[...],
                 preferred_element_type=jnp.float32) + F2b_ref[...]
    nh = nr2 + nh
    mu2 = jnp.mean(nh, axis=0, keepdims=True)
    var2 = jnp.mean((nh - mu2) * (nh - mu2), axis=0, keepdims=True)
    out_ref[...] = g2_ref[...] * (nh - mu2) / jnp.sqrt(var2 + 1e-5) \
        + b2_ref[...]


def _node_call(ov0, ov1, sp0, sp1, x, ld, dc0, dc1, Now, Nob, g1, b1,
               F1w, F1b, F2w, F2b, g2, b2):
    full = lambda a: pl.BlockSpec(a.shape, lambda: tuple(0 for _ in a.shape))
    args = (ov0, ov1, sp0, sp1, x, ld, dc0, dc1, Now, Nob, g1, b1,
            F1w, F1b, F2w, F2b, g2, b2)
    return pl.pallas_call(
        _node_body,
        in_specs=[full(a) for a in args],
        out_specs=pl.BlockSpec((N, D), lambda: (0, 0)),
        out_shape=jax.ShapeDtypeStruct((N, D), jnp.float32),
    )(*args)


def _ehf_body(ehp_ref, st_ref, g_ref, b_ref, out_ref):
    mu = st_ref[0:1, :] / E
    ex2 = st_ref[1:2, :] / E
    var = ex2 - mu * mu
    out_ref[...] = g_ref[...] * (ehp_ref[...].astype(jnp.float32) - mu) \
        * lax.rsqrt(var + 1e-5) + b_ref[...]


def _ehf_call(ehp, st, g, b):
    BE = 4000
    gr = E // BE
    return pl.pallas_call(
        _ehf_body,
        grid=(gr,),
        in_specs=[
            pl.BlockSpec((BE, D), lambda i: (i, 0)),
            pl.BlockSpec((8, D), lambda i: (0, 0)),
            pl.BlockSpec((1, D), lambda i: (0, 0)),
            pl.BlockSpec((1, D), lambda i: (0, 0)),
        ],
        out_specs=pl.BlockSpec((BE, D), lambda i: (i, 0)),
        out_shape=jax.ShapeDtypeStruct((E, D), jnp.float32),
    )(ehp, st, g, b)


def kernel(x, edge_attr, edge_index, log_deg, Qw, Qb, Kw, Kb, Ew, Eb, Vw,
           Vb, Aw, VeRow, deg_coef, Now, Nob, Eow, Eob, bn1n_g, bn1n_b,
           bn1e_g, bn1e_b, F1w, F1b, F2w, F2b, bn2_g, bn2_b):
    src2 = edge_index[0].reshape(NW * K, 1, C)
    dst2 = edge_index[1].reshape(NW * K, 1, C)

    Wc = jnp.concatenate([Qw, Kw, Vw], axis=1)
    bc = jnp.concatenate([Qb, Kb, Vb]).reshape(1, 3 * D)
    q, kv = _qkv_call(x, Wc, bc)

    kvg, qdg = _gather2_call(kv, q, src2, dst2)

    Ew4 = Ew.reshape(D, H, 2, HD)
    Ew2 = jnp.concatenate(
        [Ew4[:, :, 0, :].reshape(D, D), Ew4[:, :, 1, :].reshape(D, D)],
        axis=1)
    Eb4 = Eb.reshape(H, 2, HD)
    Eb2 = jnp.concatenate(
        [Eb4[:, 0, :].reshape(1, D), Eb4[:, 1, :].reshape(1, D)], axis=1)

    eye8 = jnp.eye(H, dtype=jnp.float32)
    Awm = (Aw[:, :, 0].T[:, :, None] * eye8[:, None, :]).reshape(D, H)
    Bm = (VeRow.transpose(1, 0, 2)[:, :, None, :]
          * eye8[:, None, :, None]).reshape(D, D)
    R = jnp.kron(eye8, jnp.ones((1, HD), jnp.float32))

    w_e, S_e, ehp, st = _edge_call(
        edge_attr, kvg, qdg, Ew2, Eb2, Awm, Bm, Eow,
        Eob.reshape(1, D), R)

    z128 = jnp.zeros((RPS, D), jnp.float32)
    sout, vout = _scatter_call(w_e, S_e, dst2, z128)

    eh = _ehf_call(ehp, st, bn1e_g.reshape(1, D), bn1e_b.reshape(1, D))

    ld128 = jnp.broadcast_to(log_deg, (N, D))
    nh = _node_call(
        vout[0, :N], vout[1, :N], sout[0, :N], sout[1, :N], x, ld128,
        deg_coef[0, :, 0].reshape(1, D), deg_coef[0, :, 1].reshape(1, D),
        Now, Nob.reshape(1, D), bn1n_g.reshape(1, D), bn1n_b.reshape(1, D),
        F1w, F1b.reshape(1, 2 * D), F2w, F2b.reshape(1, D),
        bn2_g.reshape(1, D), bn2_b.reshape(1, D))

    return nh, eh


# speedup vs baseline: 1.1011x; 1.0225x over previous
import functools

import jax
import jax.numpy as jnp
from jax import lax
from jax.experimental import pallas as pl
from jax.experimental.pallas import tpu as pltpu
from jax.experimental.pallas import tpu_sc as plsc

N = 10000
E = 320000
D = 128
H = 8
HD = 16

NC = 2
NS = 16
NW = NC * NS
C = 80
K = E // (NW * C)
NP = 10112
RPS = NP // NS


def _qkv_body(x_ref, w_ref, b_ref, q_ref, kv_ref):
    acc = jnp.dot(x_ref[...], w_ref[...],
                  preferred_element_type=jnp.float32) + b_ref[...]

    def pack(hi, lo):
        h = lax.bitcast_convert_type(
            hi.astype(jnp.bfloat16), jnp.uint16).astype(jnp.uint32) << 16
        l = lax.bitcast_convert_type(
            lo.astype(jnp.bfloat16), jnp.uint16).astype(jnp.uint32)
        return lax.bitcast_convert_type(h | l, jnp.float32)

    q_ref[...] = acc[:, :D]
    kv_ref[...] = pack(acc[:, D:2 * D], acc[:, 2 * D:])


def _qkv_call(x, Wc, bc):
    BN = 10000
    g = N // BN
    return pl.pallas_call(
        _qkv_body,
        grid=(g,),
        in_specs=[
            pl.BlockSpec((BN, D), lambda i: (i, 0)),
            pl.BlockSpec((D, 3 * D), lambda i: (0, 0)),
            pl.BlockSpec((1, 3 * D), lambda i: (0, 0)),
        ],
        out_specs=[
            pl.BlockSpec((BN, D), lambda i: (i, 0)),
            pl.BlockSpec((BN, D), lambda i: (i, 0)),
        ],
        out_shape=[jax.ShapeDtypeStruct((N, D), jnp.float32)] * 2,
    )(x, Wc, bc)


def _gather2_body(kvh, qh, src2, dst2, kvg, qdg,
                  si0, si1, di0, di1, kvb0, kvb1, qb0, qb1,
                  sx0, sx1, sg0, sg1, sw0, sw1):
    c = lax.axis_index("c")
    s = lax.axis_index("s")
    w = c * NS + s
    sis = (si0, si1)
    dis = (di0, di1)
    kvbs = (kvb0, kvb1)
    qbs = (qb0, qb1)
    sxs = (sx0, sx1)
    sgs = (sg0, sg1)
    sws = (sw0, sw1)

    def start_idx(j, b):
        r = w * K + j
        pltpu.async_copy(src2.at[r, 0], sis[b], sxs[b])
        pltpu.async_copy(dst2.at[r, 0], dis[b], sxs[b])

    def drain_idx(b):
        pltpu.make_async_copy(src2.at[w * K, 0], sis[b], sxs[b]).wait()
        pltpu.make_async_copy(src2.at[w * K, 0], dis[b], sxs[b]).wait()

    def drain_wb(b):
        pltpu.make_async_copy(kvg.at[pl.ds(0, C)], kvbs[b], sws[b]).wait()
        pltpu.make_async_copy(qdg.at[pl.ds(0, C)], qbs[b], sws[b]).wait()

    def fire(j, b):
        r = w * K + j
        drain_idx(b)

        @pl.when(j >= 2)
        def _():
            drain_wb(b)

        ck = pltpu.async_copy(kvh.at[sis[b]], kvbs[b], sgs[b])
        cq = pltpu.async_copy(qh.at[dis[b]], qbs[b], sgs[b])
        ck.wait()
        cq.wait()

        @pl.when(j + 2 < K)
        def _():
            start_idx(j + 2, b)

        base = r * C
        pltpu.async_copy(kvbs[b], kvg.at[pl.ds(base, C)], sws[b])
        pltpu.async_copy(qbs[b], qdg.at[pl.ds(base, C)], sws[b])

    start_idx(0, 0)
    start_idx(1, 1)

    def body(g, carry):
        fire(2 * g, 0)
        fire(2 * g + 1, 1)
        return carry

    lax.fori_loop(0, (K - 1) // 2, body, 0)
    fire(K - 1, (K - 1) % 2)
    drain_wb(0)
    drain_wb(1)


def _gather2_call(kvh, qh, src2, dst2):
    mesh = plsc.VectorSubcoreMesh(core_axis_name="c", subcore_axis_name="s")
    fn = functools.partial(
        pl.kernel,
        out_type=[jax.ShapeDtypeStruct((E, D), jnp.float32)] * 2,
        mesh=mesh,
        scratch_types=[
            pltpu.VMEM((C,), jnp.int32),
            pltpu.VMEM((C,), jnp.int32),
            pltpu.VMEM((C,), jnp.int32),
            pltpu.VMEM((C,), jnp.int32),
            pltpu.VMEM((C, D), jnp.float32),
            pltpu.VMEM((C, D), jnp.float32),
            pltpu.VMEM((C, D), jnp.float32),
            pltpu.VMEM((C, D), jnp.float32),
            pltpu.SemaphoreType.DMA,
            pltpu.SemaphoreType.DMA,
            pltpu.SemaphoreType.DMA,
            pltpu.SemaphoreType.DMA,
            pltpu.SemaphoreType.DMA,
            pltpu.SemaphoreType.DMA,
        ],
    )(_gather2_body)
    return fn(kvh, qh, src2, dst2)


def _edge_body(ea_ref, kv_ref, qd_ref, Ew_ref, Eb_ref, Awm_ref,
               Bm_ref, Eow_ref, Eob_ref, R_ref, w_ref, S_ref, ehp_ref,
               st_ref):
    i = pl.program_id(0)
    eav = ea_ref[...]
    f = jnp.dot(eav, Ew_ref[...], preferred_element_type=jnp.float32) \
        + Eb_ref[...]
    a = f[:, :D]
    b = f[:, D:]
    s2 = a * b
    p = jnp.sign(s2) * jnp.sqrt(jnp.abs(s2))
    def unpack(pk):
        u = lax.bitcast_convert_type(pk, jnp.uint32)
        hi = lax.bitcast_convert_type(
            (u >> 16).astype(jnp.uint16), jnp.bfloat16).astype(jnp.float32)
        lo = lax.bitcast_convert_type(
            u.astype(jnp.uint16), jnp.bfloat16).astype(jnp.float32)
        return hi, lo

    ks, vs = unpack(kv_ref[...])
    score = jax.nn.relu(ks + qd_ref[...] + p)
    logits = jnp.dot(score, Awm_ref[...], preferred_element_type=jnp.float32)
    wv = jnp.exp(jnp.clip(logits, -5.0, 5.0))
    wex = jnp.dot(wv, R_ref[...], preferred_element_type=jnp.float32)
    w_ref[...] = wex
    G = jnp.dot(score, Bm_ref[...], preferred_element_type=jnp.float32)
    S_ref[...] = (vs + G) * wex
    ehp = eav + jnp.dot(score, Eow_ref[...],
                        preferred_element_type=jnp.float32) + Eob_ref[...]
    ehp_ref[...] = ehp.astype(jnp.bfloat16)

    @pl.when(i == 0)
    def _():
        st_ref[...] = jnp.zeros_like(st_ref)

    s1 = jnp.sum(ehp, axis=0, keepdims=True)
    sq = jnp.sum(ehp * ehp, axis=0, keepdims=True)
    st_ref[...] += jnp.concatenate(
        [s1, sq, jnp.zeros((6, D), jnp.float32)], axis=0)


def _edge_call(ea, kvg, qdg, Ew2, Eb2, Awm, Bm, Eow, Eob2, R):
    BE = 8000
    g = E // BE
    full = lambda shape: pl.BlockSpec(shape, lambda i: (0, 0))
    return pl.pallas_call(
        _edge_body,
        grid=(g,),
        in_specs=[
            pl.BlockSpec((BE, D), lambda i: (i, 0)),
            pl.BlockSpec((BE, D), lambda i: (i, 0)),
            pl.BlockSpec((BE, D), lambda i: (i, 0)),
            full((D, 2 * D)),
            full((1, 2 * D)),
            full((D, H)),
            full((D, D)),
            full((D, D)),
            full((1, D)),
            full((H, D)),
        ],
        out_specs=[
            pl.BlockSpec((BE, D), lambda i: (i, 0)),
            pl.BlockSpec((BE, D), lambda i: (i, 0)),
            pl.BlockSpec((BE, D), lambda i: (i, 0)),
            pl.BlockSpec((8, D), lambda i: (0, 0)),
        ],
        out_shape=[
            jax.ShapeDtypeStruct((E, D), jnp.float32),
            jax.ShapeDtypeStruct((E, D), jnp.float32),
            jax.ShapeDtypeStruct((E, D), jnp.bfloat16),
            jax.ShapeDtypeStruct((8, D), jnp.float32),
        ],
    )(ea, kvg, qdg, Ew2, Eb2, Awm, Bm, Eow, Eob2, R)


def _scatter_body(w_hbm, S_hbm, dst2, z128, sout, vout,
                  di0, di1, sv0, sv1, acc, si0, si1, sd0, sd1):
    c = lax.axis_index("c")
    s = lax.axis_index("s")
    w = c * NS + s
    dis = (di0, di1)
    svs = (sv0, sv1)
    sis = (si0, si1)
    sds = (sd0, sd1)

    def one_pass(src_hbm, out_hbm):
        pltpu.sync_copy(z128, acc.at[pl.ds(s * RPS, RPS)])
        plsc.subcore_barrier()

        def start(j, b):
            r = w * K + j
            pltpu.async_copy(dst2.at[r, 0], dis[b], sis[b])
            pltpu.async_copy(src_hbm.at[pl.ds(r * C, C)], svs[b], sds[b])

        def fire(j, b):
            pltpu.make_async_copy(dst2.at[w * K, 0], dis[b], sis[b]).wait()
            pltpu.make_async_copy(
                src_hbm.at[pl.ds(0, C)], svs[b], sds[b]).wait()
            pltpu.sync_copy(svs[b], acc.at[dis[b]], add=True)

        start(0, 0)
        start(1, 1)

        def body(g, carry):
            for b in range(2):
                j = g + b
                fire(j, b)

                @pl.when(j + 2 < K)
                def _():
                    start(j + 2, b)
            return carry

        lax.fori_loop(0, (K - 1) // 2, lambda g, cy: body(2 * g, cy), 0)
        fire(K - 1, (K - 1) % 2)
        plsc.subcore_barrier()
        pltpu.sync_copy(acc.at[pl.ds(s * RPS, RPS)],
                        out_hbm.at[c, pl.ds(s * RPS, RPS)])
        plsc.subcore_barrier()

    one_pass(S_hbm, vout)
    one_pass(w_hbm, sout)


def _scatter_call(w_e, S_e, dst2, z128):
    mesh = plsc.VectorSubcoreMesh(core_axis_name="c", subcore_axis_name="s")
    fn = functools.partial(
        pl.kernel,
        out_type=[
            jax.ShapeDtypeStruct((NC, NP, D), jnp.float32),
            jax.ShapeDtypeStruct((NC, NP, D), jnp.float32),
        ],
        mesh=mesh,
        scratch_types=[
            pltpu.VMEM((C,), jnp.int32),
            pltpu.VMEM((C,), jnp.int32),
            pltpu.VMEM((C, D), jnp.float32),
            pltpu.VMEM((C, D), jnp.float32),
            pltpu.VMEM_SHARED((NP, D), jnp.float32),
            pltpu.SemaphoreType.DMA,
            pltpu.SemaphoreType.DMA,
            pltpu.SemaphoreType.DMA,
            pltpu.SemaphoreType.DMA,
        ],
    )(_scatter_body)
    return fn(w_e, S_e, dst2, z128)


def _node_body(ov0_ref, ov1_ref, sp0_ref, sp1_ref, x_ref, ld_ref,
               dc0_ref, dc1_ref, Now_ref, Nob_ref, g1_ref, b1_ref,
               F1w_ref, F1b_ref, F2w_ref, F2b_ref, g2_ref, b2_ref, out_ref):
    sex = 1.0 / (sp0_ref[...] + sp1_ref[...] + 1e-16)
    oV = (ov0_ref[...] + ov1_ref[...]) * sex
    nh = oV * dc0_ref[...] + (oV * ld_ref[...]) * dc1_ref[...]
    nh = jnp.dot(nh, Now_ref[...],
                 preferred_element_type=jnp.float32) + Nob_ref[...]
    nh = x_ref[...] + nh
    mu = jnp.mean(nh, axis=0, keepdims=True)
    var = jnp.mean((nh - mu) * (nh - mu), axis=0, keepdims=True)
    nh = g1_ref[...] * (nh - mu) / jnp.sqrt(var + 1e-5) + b1_ref[...]
    nr2 = nh
    hid = jax.nn.relu(
        jnp.dot(nh, F1w_ref[...], preferred_element_type=jnp.float32)
        + F1b_ref[...])
    nh = jnp.dot(hid, F2w_ref[...],
                 preferred_element_type=jnp.float32) + F2b_ref[...]
    nh = nr2 + nh
    mu2 = jnp.mean(nh, axis=0, keepdims=True)
    var2 = jnp.mean((nh - mu2) * (nh - mu2), axis=0, keepdims=True)
    out_ref[...] = g2_ref[...] * (nh - mu2) / jnp.sqrt(var2 + 1e-5) \
        + b2_ref[...]


def _node_call(ov0, ov1, sp0, sp1, x, ld, dc0, dc1, Now, Nob, g1, b1,
               F1w, F1b, F2w, F2b, g2, b2):
    full = lambda a: pl.BlockSpec(a.shape, lambda: tuple(0 for _ in a.shape))
    args = (ov0, ov1, sp0, sp1, x, ld, dc0, dc1, Now, Nob, g1, b1,
            F1w, F1b, F2w, F2b, g2, b2)
    return pl.pallas_call(
        _node_body,
        in_specs=[full(a) for a in args],
        out_specs=pl.BlockSpec((N, D), lambda: (0, 0)),
        out_shape=jax.ShapeDtypeStruct((N, D), jnp.float32),
    )(*args)


def _ehf_body(ehp_ref, st_ref, g_ref, b_ref, out_ref):
    mu = st_ref[0:1, :] / E
    ex2 = st_ref[1:2, :] / E
    var = ex2 - mu * mu
    out_ref[...] = g_ref[...] * (ehp_ref[...].astype(jnp.float32) - mu) \
        * lax.rsqrt(var + 1e-5) + b_ref[...]


def _ehf_call(ehp, st, g, b):
    BE = 8000
    gr = E // BE
    return pl.pallas_call(
        _ehf_body,
        grid=(gr,),
        in_specs=[
            pl.BlockSpec((BE, D), lambda i: (i, 0)),
            pl.BlockSpec((8, D), lambda i: (0, 0)),
            pl.BlockSpec((1, D), lambda i: (0, 0)),
            pl.BlockSpec((1, D), lambda i: (0, 0)),
        ],
        out_specs=pl.BlockSpec((BE, D), lambda i: (i, 0)),
        out_shape=jax.ShapeDtypeStruct((E, D), jnp.float32),
    )(ehp, st, g, b)


def kernel(x, edge_attr, edge_index, log_deg, Qw, Qb, Kw, Kb, Ew, Eb, Vw,
           Vb, Aw, VeRow, deg_coef, Now, Nob, Eow, Eob, bn1n_g, bn1n_b,
           bn1e_g, bn1e_b, F1w, F1b, F2w, F2b, bn2_g, bn2_b):
    src2 = edge_index[0].reshape(NW * K, 1, C)
    dst2 = edge_index[1].reshape(NW * K, 1, C)

    Wc = jnp.concatenate([Qw, Kw, Vw], axis=1)
    bc = jnp.concatenate([Qb, Kb, Vb]).reshape(1, 3 * D)
    q, kv = _qkv_call(x, Wc, bc)

    kvg, qdg = _gather2_call(kv, q, src2, dst2)

    Ew4 = Ew.reshape(D, H, 2, HD)
    Ew2 = jnp.concatenate(
        [Ew4[:, :, 0, :].reshape(D, D), Ew4[:, :, 1, :].reshape(D, D)],
        axis=1)
    Eb4 = Eb.reshape(H, 2, HD)
    Eb2 = jnp.concatenate(
        [Eb4[:, 0, :].reshape(1, D), Eb4[:, 1, :].reshape(1, D)], axis=1)

    eye8 = jnp.eye(H, dtype=jnp.float32)
    Awm = (Aw[:, :, 0].T[:, :, None] * eye8[:, None, :]).reshape(D, H)
    Bm = (VeRow.transpose(1, 0, 2)[:, :, None, :]
          * eye8[:, None, :, None]).reshape(D, D)
    R = jnp.kron(eye8, jnp.ones((1, HD), jnp.float32))

    w_e, S_e, ehp, st = _edge_call(
        edge_attr, kvg, qdg, Ew2, Eb2, Awm, Bm, Eow,
        Eob.reshape(1, D), R)

    z128 = jnp.zeros((RPS, D), jnp.float32)
    sout, vout = _scatter_call(w_e, S_e, dst2, z128)

    eh = _ehf_call(ehp, st, bn1e_g.reshape(1, D), bn1e_b.reshape(1, D))

    ld128 = jnp.broadcast_to(log_deg, (N, D))
    nh = _node_call(
        vout[0, :N], vout[1, :N], sout[0, :N], sout[1, :N], x, ld128,
        deg_coef[0, :, 0].reshape(1, D), deg_coef[0, :, 1].reshape(1, D),
        Now, Nob.reshape(1, D), bn1n_g.reshape(1, D), bn1n_b.reshape(1, D),
        F1w, F1b.reshape(1, 2 * D), F2w, F2b.reshape(1, D),
        bn2_g.reshape(1, D), bn2_b.reshape(1, D))

    return nh, eh
